# baseline (speedup 1.0000x reference)
import functools

import jax
import jax.numpy as jnp
from jax.experimental import pallas as pl
from jax.experimental.pallas import tpu as pltpu
from jax.experimental.pallas import tpu_sc as plsc

_NA = 320000
_NE = 160000
_NR = 80000
_NDUMP = 2048
_ACC = _NR + _NDUMP
_NGRP = _NA // 128


_NAP = 327680
_GPT = (_NAP // 128) // 32
_CH = 8


def _sc_scatter_body(aggr_hbm, ji_hbm, z_hbm, out_hbm,
                     ji_v, upd_v, idx_v, acc, ld0, ld1, sc0, sc1):
    c = jax.lax.axis_index("c")
    s = jax.lax.axis_index("s")
    lo = c * _NR
    share = _ACC // 16
    gpt = (_NAP // 128) // 16
    nch = gpt // _CH
    gbase = s * gpt
    iota16 = jax.lax.iota(jnp.int32, 16)
    lds = (ld0, ld1)
    scs = (sc0, sc1)

    def issue_loads(i, b, p):
        abase = (gbase + i * _CH) * 128
        pltpu.async_copy(ji_hbm.at[pl.ds(abase, _CH * 128)],
                         ji_v.at[b], lds[b])
        pltpu.async_copy(
            aggr_hbm.at[pl.ds(abase, _CH * 128), pl.ds(p * 16, 16)],
            upd_v.at[b], lds[b])

    def wait_loads(b):
        pltpu.make_async_copy(ji_hbm.at[pl.ds(0, _CH * 128)],
                              ji_v.at[b], lds[b]).wait()
        pltpu.make_async_copy(
            aggr_hbm.at[pl.ds(0, _CH * 128), pl.ds(0, 16)],
            upd_v.at[b], lds[b]).wait()

    def compute_idx(i, b):
        abase = (gbase + i * _CH) * 128

        def idxstep(k, _):
            jv = ji_v[b, pl.ds(k * 16, 16)]
            pos = abase + k * 16 + iota16
            inr = (jv >= lo) & (jv < lo + _NR)
            iv = jnp.where(inr, jv - lo, _NR + (pos & (_NDUMP - 1)))
            idx_v[b, k // 8, pl.ds((k % 8) * 16, 16)] = iv
            return 0

        jax.lax.fori_loop(0, _CH * 8, idxstep, 0)

    def issue_scatter(b):
        for r in range(_CH):
            pltpu.async_copy(upd_v.at[b, pl.ds(r * 128, 128)],
                             acc.at[idx_v.at[b, r]], scs[b], add=True)

    def wait_scatter(b):
        for r in range(_CH):
            pltpu.make_async_copy(upd_v.at[b, pl.ds(r * 128, 128)],
                                  acc.at[idx_v.at[b, r]], scs[b]).wait()

    def do_pass(p, _):
        zoff = s * share
        pltpu.sync_copy(z_hbm, acc.at[pl.ds(zoff, share)])
        plsc.subcore_barrier()

        issue_loads(0, 0, p)

        def two_chunks(t, _):
            i0 = t * 2
            wait_loads(0)
            compute_idx(i0, 0)

            @pl.when(t > 0)
            def _():
                wait_scatter(1)

            issue_loads(i0 + 1, 1, p)
            issue_scatter(0)

            wait_loads(1)
            compute_idx(i0 + 1, 1)
            wait_scatter(0)

            @pl.when(t < nch // 2 - 1)
            def _():
                issue_loads(i0 + 2, 0, p)

            issue_scatter(1)
            return 0

        jax.lax.fori_loop(0, nch // 2, two_chunks, 0)
        wait_scatter(1)

        plsc.subcore_barrier()
        foff = s * (_NR // 16)
        pltpu.sync_copy(acc.at[pl.ds(foff, _NR // 16)],
                        out_hbm.at[pl.ds(lo + foff, _NR // 16),
                                   pl.ds(p * 16, 16)])
        plsc.subcore_barrier()
        return 0

    jax.lax.fori_loop(0, 8, do_pass, 0)


def _sc_scatter(aggr, ji_idx):
    z = jnp.zeros((_ACC // 16, 16), jnp.float32)
    mesh = plsc.VectorSubcoreMesh(core_axis_name="c", subcore_axis_name="s")
    return pl.kernel(
        _sc_scatter_body,
        out_type=jax.ShapeDtypeStruct((_NE, 128), jnp.float32),
        mesh=mesh,
        compiler_params=pltpu.CompilerParams(use_tc_tiling_on_sc=False),
        scratch_types=[
            pltpu.VMEM((2, _CH * 128), jnp.int32),
            pltpu.VMEM((2, _CH * 128, 16), jnp.float32),
            pltpu.VMEM((2, _CH, 128), jnp.int32),
            pltpu.VMEM_SHARED((_ACC, 16), jnp.float32),
            pltpu.SemaphoreType.DMA,
            pltpu.SemaphoreType.DMA,
            pltpu.SemaphoreType.DMA,
            pltpu.SemaphoreType.DMA,
        ],
    )(aggr, ji_idx, z)


def _swish(x):
    return x * jax.nn.sigmoid(x)


def _k1_body(m_ref, e_ref, wm_ref, bm_ref, we_ref, g_ref):
    h = jnp.dot(m_ref[...], wm_ref[...], preferred_element_type=jnp.float32)
    h = _swish(h + bm_ref[...])
    ee = jnp.dot(e_ref[...], we_ref[...], preferred_element_type=jnp.float32)
    g_ref[...] = h * ee


def _k1(m_ji, e_rbf8, w_mkj, b_mkj, w_e8, eb):
    n = m_ji.shape[0]
    grid = (n // eb,)
    return pl.pallas_call(
        _k1_body,
        grid=grid,
        in_specs=[
            pl.BlockSpec((eb, 128), lambda i: (i, 0)),
            pl.BlockSpec((eb, 8), lambda i: (i, 0)),
            pl.BlockSpec((128, 128), lambda i: (0, 0)),
            pl.BlockSpec((1, 128), lambda i: (0, 0)),
            pl.BlockSpec((8, 128), lambda i: (0, 0)),
        ],
        out_specs=pl.BlockSpec((eb, 128), lambda i: (i, 0)),
        out_shape=jax.ShapeDtypeStruct((n, 128), jnp.float32),
    )(m_ji, e_rbf8, w_mkj, b_mkj.reshape(1, 128), w_e8)


def _k2_body(a_ref, gm_ref, wa_ref, wb_ref, o_ref):
    a = jnp.dot(a_ref[...], wa_ref[...], preferred_element_type=jnp.float32)
    gm = gm_ref[...]
    acc = jnp.dot(gm * a[:, 0:1], wb_ref[0], preferred_element_type=jnp.float32)
    for j in range(1, 8):
        acc = acc + jnp.dot(gm * a[:, j:j + 1], wb_ref[j],
                            preferred_element_type=jnp.float32)
    o_ref[...] = acc


def _k2(a_sbf, gm, w_a, w_bil2, wb):
    n = a_sbf.shape[0]
    grid = (n // wb,)
    return pl.pallas_call(
        _k2_body,
        grid=grid,
        in_specs=[
            pl.BlockSpec((wb, 48), lambda i: (i, 0)),
            pl.BlockSpec((wb, 128), lambda i: (i, 0)),
            pl.BlockSpec((48, 8), lambda i: (0, 0)),
            pl.BlockSpec((8, 128, 128), lambda i: (0, 0, 0)),
        ],
        out_specs=pl.BlockSpec((wb, 128), lambda i: (i, 0)),
        out_shape=jax.ShapeDtypeStruct((n, 128), jnp.float32),
    )(a_sbf, gm, w_a, w_bil2)


def _k3_body(m_ref, d_ref, w_ref, b_ref, o_ref):
    m = m_ref[...]
    x = d_ref[...] + _swish(
        jnp.dot(m, w_ref[0], preferred_element_type=jnp.float32) + b_ref[0, 0])
    r = _swish(jnp.dot(x, w_ref[1], preferred_element_type=jnp.float32) + b_ref[0, 1])
    r = _swish(jnp.dot(r, w_ref[2], preferred_element_type=jnp.float32) + b_ref[0, 2])
    x = r + x
    x = _swish(jnp.dot(x, w_ref[3], preferred_element_type=jnp.float32) + b_ref[0, 3]) + m
    r = _swish(jnp.dot(x, w_ref[4], preferred_element_type=jnp.float32) + b_ref[0, 4])
    r = _swish(jnp.dot(r, w_ref[5], preferred_element_type=jnp.float32) + b_ref[0, 5])
    x = r + x
    r = _swish(jnp.dot(x, w_ref[6], preferred_element_type=jnp.float32) + b_ref[0, 6])
    r = _swish(jnp.dot(r, w_ref[7], preferred_element_type=jnp.float32) + b_ref[0, 7])
    o_ref[...] = r + x


def _k3(m_ji, directed, ws, bs, eb):
    n = m_ji.shape[0]
    grid = (n // eb,)
    return pl.pallas_call(
        _k3_body,
        grid=grid,
        in_specs=[
            pl.BlockSpec((eb, 128), lambda i: (i, 0)),
            pl.BlockSpec((eb, 128), lambda i: (i, 0)),
            pl.BlockSpec((8, 128, 128), lambda i: (0, 0, 0)),
            pl.BlockSpec((1, 8, 128), lambda i: (0, 0, 0)),
        ],
        out_specs=pl.BlockSpec((eb, 128), lambda i: (i, 0)),
        out_shape=jax.ShapeDtypeStruct((n, 128), jnp.float32),
    )(m_ji, directed, ws, bs)


def kernel(m_ji, nbr_list, angle_list, e_rbf, a_sbf, kj_idx, ji_idx,
           w_mkj, b_mkj, w_e, w_a, w_bil,
           res0_w0, res0_b0, res0_w1, res0_b1,
           res1_w0, res1_b0, res1_w1, res1_b1,
           res2_w0, res2_b0, res2_w1, res2_b1,
           w_mji, b_mji, w_post, b_post):
    n_edges = m_ji.shape[0]

    e_rbf8 = jnp.pad(e_rbf, ((0, 0), (0, 2)))
    w_e8 = jnp.pad(w_e, ((0, 2), (0, 0)))
    a_sbf48 = jnp.pad(a_sbf, ((0, 0), (0, 6)))
    w_a48 = jnp.pad(w_a, ((0, 6), (0, 0)))
    w_bil2 = jnp.transpose(w_bil, (1, 2, 0))

    g = _k1(m_ji, e_rbf8, w_mkj, b_mkj, w_e8, eb=1600)

    npad = _NAP - a_sbf.shape[0]
    kj_p = jnp.pad(kj_idx, (0, npad))
    ji_p = jnp.pad(ji_idx, (0, npad), constant_values=2**30)
    a_sbf48p = jnp.pad(a_sbf48, ((0, npad), (0, 0)))
    gm = jnp.take(g, kj_p, axis=0)
    aggr = _k2(a_sbf48p, gm, w_a48, w_bil2, wb=1280)
    directed = _sc_scatter(aggr, ji_p)

    ws = jnp.stack([w_mji, res0_w0, res0_w1, w_post,
                    res1_w0, res1_w1, res2_w0, res2_w1])
    bs = jnp.stack([b_mji, res0_b0, res0_b1, b_post,
                    res1_b0, res1_b1, res2_b0, res2_b1]).reshape(1, 8, 128)
    return _k3(m_ji, directed, ws, bs, eb=1600)


# speedup vs baseline: 1.2121x; 1.2121x over previous
import functools

import jax
import jax.numpy as jnp
from jax.experimental import pallas as pl
from jax.experimental.pallas import tpu as pltpu
from jax.experimental.pallas import tpu_sc as plsc

_NA = 320000
_NE = 160000
_NR = 80000
_NDUMP = 2048
_ACC = _NR + _NDUMP
_NGRP = _NA // 128


_NAP = 327680
_GPT = (_NAP // 128) // 32
_CH = 8


def _sc_scatter_body(aggr_hbm, ji_hbm, z_hbm, out_hbm,
                     ji_v, upd_v, idx_v, acc, ld0, ld1, sc0, sc1):
    c = jax.lax.axis_index("c")
    s = jax.lax.axis_index("s")
    lo = c * _NR
    share = _ACC // 16
    gpt = (_NAP // 128) // 16
    nch = gpt // _CH
    gbase = s * gpt
    iota16 = jax.lax.iota(jnp.int32, 16)
    lds = (ld0, ld1)
    scs = (sc0, sc1)

    def issue_loads(i, b, p):
        abase = (gbase + i * _CH) * 128
        pltpu.async_copy(ji_hbm.at[pl.ds(abase, _CH * 128)],
                         ji_v.at[b], lds[b])
        pltpu.async_copy(
            aggr_hbm.at[pl.ds(abase, _CH * 128), pl.ds(p * 16, 16)],
            upd_v.at[b], lds[b])

    def wait_loads(b):
        pltpu.make_async_copy(ji_hbm.at[pl.ds(0, _CH * 128)],
                              ji_v.at[b], lds[b]).wait()
        pltpu.make_async_copy(
            aggr_hbm.at[pl.ds(0, _CH * 128), pl.ds(0, 16)],
            upd_v.at[b], lds[b]).wait()

    def compute_idx(i, b):
        abase = (gbase + i * _CH) * 128

        def idxstep(k, _):
            jv = ji_v[b, pl.ds(k * 16, 16)]
            pos = abase + k * 16 + iota16
            inr = (jv >= lo) & (jv < lo + _NR)
            iv = jnp.where(inr, jv - lo, _NR + (pos & (_NDUMP - 1)))
            idx_v[b, k // 8, pl.ds((k % 8) * 16, 16)] = iv
            return 0

        jax.lax.fori_loop(0, _CH * 8, idxstep, 0)

    def issue_scatter(b):
        for r in range(_CH):
            pltpu.async_copy(upd_v.at[b, pl.ds(r * 128, 128)],
                             acc.at[idx_v.at[b, r]], scs[b], add=True)

    def wait_scatter(b):
        for r in range(_CH):
            pltpu.make_async_copy(upd_v.at[b, pl.ds(r * 128, 128)],
                                  acc.at[idx_v.at[b, r]], scs[b]).wait()

    def do_pass(p, _):
        zoff = s * share
        pltpu.sync_copy(z_hbm, acc.at[pl.ds(zoff, share)])
        plsc.subcore_barrier()

        issue_loads(0, 0, p)

        def two_chunks(t, _):
            i0 = t * 2
            wait_loads(0)
            compute_idx(i0, 0)

            @pl.when(t > 0)
            def _():
                wait_scatter(1)

            issue_loads(i0 + 1, 1, p)
            issue_scatter(0)

            wait_loads(1)
            compute_idx(i0 + 1, 1)
            wait_scatter(0)

            @pl.when(t < nch // 2 - 1)
            def _():
                issue_loads(i0 + 2, 0, p)

            issue_scatter(1)
            return 0

        jax.lax.fori_loop(0, nch // 2, two_chunks, 0)
        wait_scatter(1)

        plsc.subcore_barrier()
        foff = s * (_NR // 16)
        pltpu.sync_copy(acc.at[pl.ds(foff, _NR // 16)],
                        out_hbm.at[pl.ds(lo + foff, _NR // 16),
                                   pl.ds(p * 16, 16)])
        plsc.subcore_barrier()
        return 0

    jax.lax.fori_loop(0, 8, do_pass, 0)


def _sc_scatter(aggr, ji_idx):
    z = jnp.zeros((_ACC // 16, 16), jnp.float32)
    mesh = plsc.VectorSubcoreMesh(core_axis_name="c", subcore_axis_name="s")
    return pl.kernel(
        _sc_scatter_body,
        out_type=jax.ShapeDtypeStruct((_NE, 128), jnp.float32),
        mesh=mesh,
        compiler_params=pltpu.CompilerParams(use_tc_tiling_on_sc=False),
        scratch_types=[
            pltpu.VMEM((2, _CH * 128), jnp.int32),
            pltpu.VMEM((2, _CH * 128, 16), jnp.float32),
            pltpu.VMEM((2, _CH, 128), jnp.int32),
            pltpu.VMEM_SHARED((_ACC, 16), jnp.float32),
            pltpu.SemaphoreType.DMA,
            pltpu.SemaphoreType.DMA,
            pltpu.SemaphoreType.DMA,
            pltpu.SemaphoreType.DMA,
        ],
    )(aggr, ji_idx, z)


def _swish(x):
    return x * jax.nn.sigmoid(x)


def _k1_body(m_ref, e_ref, wm_ref, bm_ref, we_ref, g_ref):
    h = jnp.dot(m_ref[...], wm_ref[...], preferred_element_type=jnp.float32)
    h = _swish(h + bm_ref[...])
    ee = jnp.dot(e_ref[...], we_ref[...], preferred_element_type=jnp.float32)
    g_ref[...] = h * ee


def _k1(m_ji, e_rbf8, w_mkj, b_mkj, w_e8, eb):
    n = m_ji.shape[0]
    grid = (n // eb,)
    return pl.pallas_call(
        _k1_body,
        grid=grid,
        in_specs=[
            pl.BlockSpec((eb, 128), lambda i: (i, 0)),
            pl.BlockSpec((eb, 8), lambda i: (i, 0)),
            pl.BlockSpec((128, 128), lambda i: (0, 0)),
            pl.BlockSpec((1, 128), lambda i: (0, 0)),
            pl.BlockSpec((8, 128), lambda i: (0, 0)),
        ],
        out_specs=pl.BlockSpec((eb, 128), lambda i: (i, 0)),
        out_shape=jax.ShapeDtypeStruct((n, 128), jnp.float32),
    )(m_ji, e_rbf8, w_mkj, b_mkj.reshape(1, 128), w_e8)


def _k2_body(a_ref, gm_ref, wa_ref, wb_ref, o_ref):
    a = jnp.dot(a_ref[...], wa_ref[...], preferred_element_type=jnp.float32)
    gm = gm_ref[...]
    acc = jnp.dot(gm * a[:, 0:1], wb_ref[0], preferred_element_type=jnp.float32)
    for j in range(1, 8):
        acc = acc + jnp.dot(gm * a[:, j:j + 1], wb_ref[j],
                            preferred_element_type=jnp.float32)
    o_ref[...] = acc


def _k2(a_sbf, gm, w_a, w_bil2, wb):
    n = a_sbf.shape[0]
    grid = (n // wb,)
    gmax = gm.shape[0] // wb - 1
    return pl.pallas_call(
        _k2_body,
        grid=grid,
        in_specs=[
            pl.BlockSpec((wb, 48), lambda i: (i, 0)),
            pl.BlockSpec((wb, 128), lambda i: (jnp.minimum(i, gmax), 0)),
            pl.BlockSpec((48, 8), lambda i: (0, 0)),
            pl.BlockSpec((8, 128, 128), lambda i: (0, 0, 0)),
        ],
        out_specs=pl.BlockSpec((wb, 128), lambda i: (i, 0)),
        out_shape=jax.ShapeDtypeStruct((n, 128), jnp.float32),
    )(a_sbf, gm, w_a, w_bil2)


def _k3_body(m_ref, d_ref, w_ref, b_ref, o_ref):
    m = m_ref[...]
    x = d_ref[...] + _swish(
        jnp.dot(m, w_ref[0], preferred_element_type=jnp.float32) + b_ref[0, 0])
    r = _swish(jnp.dot(x, w_ref[1], preferred_element_type=jnp.float32) + b_ref[0, 1])
    r = _swish(jnp.dot(r, w_ref[2], preferred_element_type=jnp.float32) + b_ref[0, 2])
    x = r + x
    x = _swish(jnp.dot(x, w_ref[3], preferred_element_type=jnp.float32) + b_ref[0, 3]) + m
    r = _swish(jnp.dot(x, w_ref[4], preferred_element_type=jnp.float32) + b_ref[0, 4])
    r = _swish(jnp.dot(r, w_ref[5], preferred_element_type=jnp.float32) + b_ref[0, 5])
    x = r + x
    r = _swish(jnp.dot(x, w_ref[6], preferred_element_type=jnp.float32) + b_ref[0, 6])
    r = _swish(jnp.dot(r, w_ref[7], preferred_element_type=jnp.float32) + b_ref[0, 7])
    o_ref[...] = r + x


def _k3(m_ji, directed, ws, bs, eb):
    n = m_ji.shape[0]
    grid = (n // eb,)
    return pl.pallas_call(
        _k3_body,
        grid=grid,
        in_specs=[
            pl.BlockSpec((eb, 128), lambda i: (i, 0)),
            pl.BlockSpec((eb, 128), lambda i: (i, 0)),
            pl.BlockSpec((8, 128, 128), lambda i: (0, 0, 0)),
            pl.BlockSpec((1, 8, 128), lambda i: (0, 0, 0)),
        ],
        out_specs=pl.BlockSpec((eb, 128), lambda i: (i, 0)),
        out_shape=jax.ShapeDtypeStruct((n, 128), jnp.float32),
    )(m_ji, directed, ws, bs)


def kernel(m_ji, nbr_list, angle_list, e_rbf, a_sbf, kj_idx, ji_idx,
           w_mkj, b_mkj, w_e, w_a, w_bil,
           res0_w0, res0_b0, res0_w1, res0_b1,
           res1_w0, res1_b0, res1_w1, res1_b1,
           res2_w0, res2_b0, res2_w1, res2_b1,
           w_mji, b_mji, w_post, b_post):
    n_edges = m_ji.shape[0]

    e_rbf8 = jnp.pad(e_rbf, ((0, 0), (0, 2)))
    w_e8 = jnp.pad(w_e, ((0, 2), (0, 0)))
    a_sbf48 = jnp.pad(a_sbf, ((0, 0), (0, 6)))
    w_a48 = jnp.pad(w_a, ((0, 6), (0, 0)))
    w_bil2 = jnp.transpose(w_bil, (1, 2, 0))

    g = _k1(m_ji, e_rbf8, w_mkj, b_mkj, w_e8, eb=1600)

    npad = _NAP - a_sbf.shape[0]
    ji_p = jnp.pad(ji_idx, (0, npad), constant_values=2**30)
    a_sbf48p = jnp.pad(a_sbf48, ((0, npad), (0, 0)))
    gm = jnp.take(g, kj_idx, axis=0)
    aggr = _k2(a_sbf48p, gm, w_a48, w_bil2, wb=1280)
    directed = _sc_scatter(aggr, ji_p)

    ws = jnp.stack([w_mji, res0_w0, res0_w1, w_post,
                    res1_w0, res1_w1, res2_w0, res2_w1])
    bs = jnp.stack([b_mji, res0_b0, res0_b1, b_post,
                    res1_b0, res1_b1, res2_b0, res2_b1]).reshape(1, 8, 128)
    return _k3(m_ji, directed, ws, bs, eb=1600)


# speedup vs baseline: 1.3220x; 1.0906x over previous
import functools

import jax
import jax.numpy as jnp
from jax.experimental import pallas as pl
from jax.experimental.pallas import tpu as pltpu
from jax.experimental.pallas import tpu_sc as plsc

_NA = 320000
_NE = 160000
_NR = 80000
_NDUMP = 2048
_ACC = _NR + _NDUMP
_NGRP = _NA // 128


_NAP = 327680
_GPT = (_NAP // 128) // 32
_CH = 8


def _sc_scatter_body(aggr_hbm, ji_hbm, z_hbm, out_hbm,
                     ji_v, upd_v, idx_v, acc, ld0, ld1, sc0, sc1):
    c = jax.lax.axis_index("c")
    s = jax.lax.axis_index("s")
    lo = c * _NR
    share = _ACC // 16
    gpt = (_NAP // 128) // 16
    nch = gpt // _CH
    gbase = s * gpt
    iota16 = jax.lax.iota(jnp.int32, 16)
    lds = (ld0, ld1)
    scs = (sc0, sc1)

    def issue_loads(i, b, p):
        abase = (gbase + i * _CH) * 128
        pltpu.async_copy(ji_hbm.at[pl.ds(abase, _CH * 128)],
                         ji_v.at[b], lds[b])
        pltpu.async_copy(
            aggr_hbm.at[pl.ds(abase, _CH * 128), pl.ds(p * 16, 16)],
            upd_v.at[b], lds[b])

    def wait_loads(b):
        pltpu.make_async_copy(ji_hbm.at[pl.ds(0, _CH * 128)],
                              ji_v.at[b], lds[b]).wait()
        pltpu.make_async_copy(
            aggr_hbm.at[pl.ds(0, _CH * 128), pl.ds(0, 16)],
            upd_v.at[b], lds[b]).wait()

    def compute_idx(i, b):
        abase = (gbase + i * _CH) * 128

        def idxstep(k, _):
            jv = ji_v[b, pl.ds(k * 16, 16)]
            pos = abase + k * 16 + iota16
            inr = (jv >= lo) & (jv < lo + _NR)
            iv = jnp.where(inr, jv - lo, _NR + (pos & (_NDUMP - 1)))
            idx_v[b, k // 8, pl.ds((k % 8) * 16, 16)] = iv
            return 0

        jax.lax.fori_loop(0, _CH * 8, idxstep, 0)

    def issue_scatter(b):
        for r in range(_CH):
            pltpu.async_copy(upd_v.at[b, pl.ds(r * 128, 128)],
                             acc.at[idx_v.at[b, r]], scs[b], add=True)

    def wait_scatter(b):
        for r in range(_CH):
            pltpu.make_async_copy(upd_v.at[b, pl.ds(r * 128, 128)],
                                  acc.at[idx_v.at[b, r]], scs[b]).wait()

    def do_pass(p, _):
        zoff = s * share
        pltpu.sync_copy(z_hbm, acc.at[pl.ds(zoff, share)])
        plsc.subcore_barrier()

        issue_loads(0, 0, p)

        def two_chunks(t, _):
            i0 = t * 2
            wait_loads(0)
            compute_idx(i0, 0)

            @pl.when(t > 0)
            def _():
                wait_scatter(1)

            issue_loads(i0 + 1, 1, p)
            issue_scatter(0)

            wait_loads(1)
            compute_idx(i0 + 1, 1)
            wait_scatter(0)

            @pl.when(t < nch // 2 - 1)
            def _():
                issue_loads(i0 + 2, 0, p)

            issue_scatter(1)
            return 0

        jax.lax.fori_loop(0, nch // 2, two_chunks, 0)
        wait_scatter(1)

        plsc.subcore_barrier()
        foff = s * (_NR // 16)
        pltpu.sync_copy(acc.at[pl.ds(foff, _NR // 16)],
                        out_hbm.at[pl.ds(lo + foff, _NR // 16),
                                   pl.ds(p * 16, 16)])
        plsc.subcore_barrier()
        return 0

    jax.lax.fori_loop(0, 8, do_pass, 0)


def _sc_scatter(aggr, ji_idx):
    z = jnp.zeros((_ACC // 16, 16), jnp.float32)
    mesh = plsc.VectorSubcoreMesh(core_axis_name="c", subcore_axis_name="s")
    return pl.kernel(
        _sc_scatter_body,
        out_type=jax.ShapeDtypeStruct((_NE, 128), jnp.float32),
        mesh=mesh,
        compiler_params=pltpu.CompilerParams(use_tc_tiling_on_sc=False),
        scratch_types=[
            pltpu.VMEM((2, _CH * 128), jnp.int32),
            pltpu.VMEM((2, _CH * 128, 16), jnp.float32),
            pltpu.VMEM((2, _CH, 128), jnp.int32),
            pltpu.VMEM_SHARED((_ACC, 16), jnp.float32),
            pltpu.SemaphoreType.DMA,
            pltpu.SemaphoreType.DMA,
            pltpu.SemaphoreType.DMA,
            pltpu.SemaphoreType.DMA,
        ],
    )(aggr, ji_idx, z)


def _sc_gather_body(g_hbm, kj_hbm, gm_hbm, idx_v, rows_v, lsem, g0, g1, s0, s1):
    c = jax.lax.axis_index("c")
    s = jax.lax.axis_index("s")
    w = s * 2 + c
    base = w * (_NA // 32)
    gs = (g0, g1)
    ss = (s0, s1)
    nch = (_NA // 32) // 400

    def idx_load(i, b):
        pltpu.sync_copy(kj_hbm.at[pl.ds(base + i * 400, 400)], idx_v.at[b])

    def gather_start(b):
        pltpu.async_copy(g_hbm.at[idx_v.at[b]], rows_v.at[b], gs[b])

    def gather_wait(b):
        pltpu.make_async_copy(g_hbm.at[idx_v.at[b]], rows_v.at[b], gs[b]).wait()

    def store_start(i, b):
        pltpu.async_copy(rows_v.at[b], gm_hbm.at[pl.ds(base + i * 400, 400)],
                         ss[b])

    def store_wait(b):
        pltpu.make_async_copy(rows_v.at[b], gm_hbm.at[pl.ds(0, 400)],
                              ss[b]).wait()

    idx_load(0, 0)
    gather_start(0)

    def chunk2(t, _):
        i0 = t * 2
        idx_load(i0 + 1, 1)

        @pl.when(t > 0)
        def _():
            store_wait(1)

        gather_start(1)
        gather_wait(0)
        store_start(i0, 0)

        @pl.when(t < nch // 2)
        def _():
            idx_load(i0 + 2, 0)
            store_wait(0)
            gather_start(0)

        gather_wait(1)
        store_start(i0 + 1, 1)
        return 0

    jax.lax.fori_loop(0, nch // 2, chunk2, 0)
    gather_wait(0)
    store_wait(1)
    store_start(nch - 1, 0)
    store_wait(0)


def _sc_gather(g, kj_idx):
    mesh = plsc.VectorSubcoreMesh(core_axis_name="c", subcore_axis_name="s")
    return pl.kernel(
        _sc_gather_body,
        out_type=jax.ShapeDtypeStruct((_NA, 128), jnp.float32),
        mesh=mesh,
        compiler_params=pltpu.CompilerParams(use_tc_tiling_on_sc=False),
        scratch_types=[
            pltpu.VMEM((2, 400), jnp.int32),
            pltpu.VMEM((2, 400, 128), jnp.float32),
            pltpu.SemaphoreType.DMA,
            pltpu.SemaphoreType.DMA,
            pltpu.SemaphoreType.DMA,
            pltpu.SemaphoreType.DMA,
            pltpu.SemaphoreType.DMA,
        ],
    )(g, kj_idx)


def _swish(x):
    return x * jax.nn.sigmoid(x)


def _k1_body(m_ref, e_ref, wm_ref, bm_ref, we_ref, g_ref):
    h = jnp.dot(m_ref[...], wm_ref[...], preferred_element_type=jnp.float32)
    h = _swish(h + bm_ref[...])
    ee = jnp.dot(e_ref[...], we_ref[...], preferred_element_type=jnp.float32)
    g_ref[...] = h * ee


def _k1(m_ji, e_rbf8, w_mkj, b_mkj, w_e8, eb):
    n = m_ji.shape[0]
    grid = (n // eb,)
    return pl.pallas_call(
        _k1_body,
        grid=grid,
        in_specs=[
            pl.BlockSpec((eb, 128), lambda i: (i, 0)),
            pl.BlockSpec((eb, 8), lambda i: (i, 0)),
            pl.BlockSpec((128, 128), lambda i: (0, 0)),
            pl.BlockSpec((1, 128), lambda i: (0, 0)),
            pl.BlockSpec((8, 128), lambda i: (0, 0)),
        ],
        out_specs=pl.BlockSpec((eb, 128), lambda i: (i, 0)),
        out_shape=jax.ShapeDtypeStruct((n, 128), jnp.float32),
    )(m_ji, e_rbf8, w_mkj, b_mkj.reshape(1, 128), w_e8)


def _k2_body(a_ref, gm_ref, wa_ref, wb_ref, o_ref):
    a = jnp.dot(a_ref[...], wa_ref[...], preferred_element_type=jnp.float32)
    gm = gm_ref[...]
    acc = jnp.dot(gm * a[:, 0:1], wb_ref[0], preferred_element_type=jnp.float32)
    for j in range(1, 8):
        acc = acc + jnp.dot(gm * a[:, j:j + 1], wb_ref[j],
                            preferred_element_type=jnp.float32)
    o_ref[...] = acc


def _k2(a_sbf, gm, w_a, w_bil2, wb):
    n = a_sbf.shape[0]
    grid = (n // wb,)
    gmax = gm.shape[0] // wb - 1
    return pl.pallas_call(
        _k2_body,
        grid=grid,
        in_specs=[
            pl.BlockSpec((wb, 48), lambda i: (i, 0)),
            pl.BlockSpec((wb, 128), lambda i: (jnp.minimum(i, gmax), 0)),
            pl.BlockSpec((48, 8), lambda i: (0, 0)),
            pl.BlockSpec((8, 128, 128), lambda i: (0, 0, 0)),
        ],
        out_specs=pl.BlockSpec((wb, 128), lambda i: (i, 0)),
        out_shape=jax.ShapeDtypeStruct((n, 128), jnp.float32),
    )(a_sbf, gm, w_a, w_bil2)


def _k3_body(m_ref, d_ref, w_ref, b_ref, o_ref):
    m = m_ref[...]
    x = d_ref[...] + _swish(
        jnp.dot(m, w_ref[0], preferred_element_type=jnp.float32) + b_ref[0, 0])
    r = _swish(jnp.dot(x, w_ref[1], preferred_element_type=jnp.float32) + b_ref[0, 1])
    r = _swish(jnp.dot(r, w_ref[2], preferred_element_type=jnp.float32) + b_ref[0, 2])
    x = r + x
    x = _swish(jnp.dot(x, w_ref[3], preferred_element_type=jnp.float32) + b_ref[0, 3]) + m
    r = _swish(jnp.dot(x, w_ref[4], preferred_element_type=jnp.float32) + b_ref[0, 4])
    r = _swish(jnp.dot(r, w_ref[5], preferred_element_type=jnp.float32) + b_ref[0, 5])
    x = r + x
    r = _swish(jnp.dot(x, w_ref[6], preferred_element_type=jnp.float32) + b_ref[0, 6])
    r = _swish(jnp.dot(r, w_ref[7], preferred_element_type=jnp.float32) + b_ref[0, 7])
    o_ref[...] = r + x


def _k3(m_ji, directed, ws, bs, eb):
    n = m_ji.shape[0]
    grid = (n // eb,)
    return pl.pallas_call(
        _k3_body,
        grid=grid,
        in_specs=[
            pl.BlockSpec((eb, 128), lambda i: (i, 0)),
            pl.BlockSpec((eb, 128), lambda i: (i, 0)),
            pl.BlockSpec((8, 128, 128), lambda i: (0, 0, 0)),
            pl.BlockSpec((1, 8, 128), lambda i: (0, 0, 0)),
        ],
        out_specs=pl.BlockSpec((eb, 128), lambda i: (i, 0)),
        out_shape=jax.ShapeDtypeStruct((n, 128), jnp.float32),
    )(m_ji, directed, ws, bs)


def kernel(m_ji, nbr_list, angle_list, e_rbf, a_sbf, kj_idx, ji_idx,
           w_mkj, b_mkj, w_e, w_a, w_bil,
           res0_w0, res0_b0, res0_w1, res0_b1,
           res1_w0, res1_b0, res1_w1, res1_b1,
           res2_w0, res2_b0, res2_w1, res2_b1,
           w_mji, b_mji, w_post, b_post):
    n_edges = m_ji.shape[0]

    e_rbf8 = jnp.pad(e_rbf, ((0, 0), (0, 2)))
    w_e8 = jnp.pad(w_e, ((0, 2), (0, 0)))
    a_sbf48 = jnp.pad(a_sbf, ((0, 0), (0, 6)))
    w_a48 = jnp.pad(w_a, ((0, 6), (0, 0)))
    w_bil2 = jnp.transpose(w_bil, (1, 2, 0))

    g = _k1(m_ji, e_rbf8, w_mkj, b_mkj, w_e8, eb=1600)

    npad = _NAP - a_sbf.shape[0]
    ji_p = jnp.pad(ji_idx, (0, npad), constant_values=2**30)
    a_sbf48p = jnp.pad(a_sbf48, ((0, npad), (0, 0)))
    gm = _sc_gather(g, kj_idx)
    aggr = _k2(a_sbf48p, gm, w_a48, w_bil2, wb=1280)
    directed = _sc_scatter(aggr, ji_p)

    ws = jnp.stack([w_mji, res0_w0, res0_w1, w_post,
                    res1_w0, res1_w1, res2_w0, res2_w1])
    bs = jnp.stack([b_mji, res0_b0, res0_b1, b_post,
                    res1_b0, res1_b1, res2_b0, res2_b1]).reshape(1, 8, 128)
    return _k3(m_ji, directed, ws, bs, eb=1600)


# speedup vs baseline: 1.3770x; 1.0416x over previous
import functools

import jax
import jax.numpy as jnp
from jax.experimental import pallas as pl
from jax.experimental.pallas import tpu as pltpu
from jax.experimental.pallas import tpu_sc as plsc

_NA = 320000
_NE = 160000
_NR = 80000
_NDUMP = 2048
_ACC = _NR + _NDUMP
_NGRP = _NA // 128


_NAP = 327680
_GPT = (_NAP // 128) // 32
_CH = 8


def _sc_scatter_body(aggr_hbm, ji_hbm, z_hbm, out_hbm,
                     ji_v, upd_v, idx_v, acc, ld0, ld1, sc0, sc1):
    c = jax.lax.axis_index("c")
    s = jax.lax.axis_index("s")
    lo = c * _NR
    share = _ACC // 16
    gpt = (_NAP // 128) // 16
    nch = gpt // _CH
    gbase = s * gpt
    iota16 = jax.lax.iota(jnp.int32, 16)
    lds = (ld0, ld1)
    scs = (sc0, sc1)

    def issue_loads(i, b, p):
        abase = (gbase + i * _CH) * 128
        pltpu.async_copy(ji_hbm.at[pl.ds(abase, _CH * 128)],
                         ji_v.at[b], lds[b])
        pltpu.async_copy(
            aggr_hbm.at[pl.ds(abase, _CH * 128), pl.ds(p * 16, 16)],
            upd_v.at[b], lds[b])

    def wait_loads(b):
        pltpu.make_async_copy(ji_hbm.at[pl.ds(0, _CH * 128)],
                              ji_v.at[b], lds[b]).wait()
        pltpu.make_async_copy(
            aggr_hbm.at[pl.ds(0, _CH * 128), pl.ds(0, 16)],
            upd_v.at[b], lds[b]).wait()

    def compute_idx(i, b):
        abase = (gbase + i * _CH) * 128

        def idxstep(k, _):
            jv = ji_v[b, pl.ds(k * 16, 16)]
            pos = abase + k * 16 + iota16
            inr = (jv >= lo) & (jv < lo + _NR)
            iv = jnp.where(inr, jv - lo, _NR + (pos & (_NDUMP - 1)))
            idx_v[b, k // 8, pl.ds((k % 8) * 16, 16)] = iv
            return 0

        jax.lax.fori_loop(0, _CH * 8, idxstep, 0)

    def issue_scatter(b):
        for r in range(_CH):
            pltpu.async_copy(upd_v.at[b, pl.ds(r * 128, 128)],
                             acc.at[idx_v.at[b, r]], scs[b], add=True)

    def wait_scatter(b):
        for r in range(_CH):
            pltpu.make_async_copy(upd_v.at[b, pl.ds(r * 128, 128)],
                                  acc.at[idx_v.at[b, r]], scs[b]).wait()

    def do_pass(p, _):
        zoff = s * share
        pltpu.sync_copy(z_hbm, acc.at[pl.ds(zoff, share)])
        plsc.subcore_barrier()

        issue_loads(0, 0, p)

        def two_chunks(t, _):
            i0 = t * 2
            wait_loads(0)
            compute_idx(i0, 0)

            @pl.when(t > 0)
            def _():
                wait_scatter(1)

            issue_loads(i0 + 1, 1, p)
            issue_scatter(0)

            wait_loads(1)
            compute_idx(i0 + 1, 1)
            wait_scatter(0)

            @pl.when(t < nch // 2 - 1)
            def _():
                issue_loads(i0 + 2, 0, p)

            issue_scatter(1)
            return 0

        jax.lax.fori_loop(0, nch // 2, two_chunks, 0)
        wait_scatter(1)

        plsc.subcore_barrier()
        foff = s * (_NR // 16)
        pltpu.sync_copy(acc.at[pl.ds(foff, _NR // 16)],
                        out_hbm.at[pl.ds(lo + foff, _NR // 16),
                                   pl.ds(p * 16, 16)])
        plsc.subcore_barrier()
        return 0

    jax.lax.fori_loop(0, 8, do_pass, 0)


def _sc_scatter(aggr, ji_idx):
    z = jnp.zeros((_ACC // 16, 16), jnp.float32)
    mesh = plsc.VectorSubcoreMesh(core_axis_name="c", subcore_axis_name="s")
    return pl.kernel(
        _sc_scatter_body,
        out_type=jax.ShapeDtypeStruct((_NE, 128), jnp.float32),
        mesh=mesh,
        compiler_params=pltpu.CompilerParams(use_tc_tiling_on_sc=False),
        scratch_types=[
            pltpu.VMEM((2, _CH * 128), jnp.int32),
            pltpu.VMEM((2, _CH * 128, 16), jnp.float32),
            pltpu.VMEM((2, _CH, 128), jnp.int32),
            pltpu.VMEM_SHARED((_ACC, 16), jnp.float32),
            pltpu.SemaphoreType.DMA,
            pltpu.SemaphoreType.DMA,
            pltpu.SemaphoreType.DMA,
            pltpu.SemaphoreType.DMA,
        ],
    )(aggr, ji_idx, z)


def _sc_gather_body(g_hbm, kj_hbm, gm_hbm, idx_v, rows_v, lsem, g0, g1, s0, s1):
    c = jax.lax.axis_index("c")
    s = jax.lax.axis_index("s")
    w = s * 2 + c
    base = w * (_NA // 32)
    gs = (g0, g1)
    ss = (s0, s1)
    nch = (_NA // 32) // 400

    def idx_load(i, b):
        pltpu.sync_copy(kj_hbm.at[pl.ds(base + i * 400, 400)], idx_v.at[b])

    def gather_start(b):
        pltpu.async_copy(g_hbm.at[idx_v.at[b]], rows_v.at[b], gs[b])

    def gather_wait(b):
        pltpu.make_async_copy(g_hbm.at[idx_v.at[b]], rows_v.at[b], gs[b]).wait()

    def store_start(i, b):
        pltpu.async_copy(rows_v.at[b], gm_hbm.at[pl.ds(base + i * 400, 400)],
                         ss[b])

    def store_wait(b):
        pltpu.make_async_copy(rows_v.at[b], gm_hbm.at[pl.ds(0, 400)],
                              ss[b]).wait()

    idx_load(0, 0)
    gather_start(0)

    def chunk2(t, _):
        i0 = t * 2
        idx_load(i0 + 1, 1)

        @pl.when(t > 0)
        def _():
            store_wait(1)

        gather_start(1)
        gather_wait(0)
        store_start(i0, 0)

        @pl.when(t < nch // 2)
        def _():
            idx_load(i0 + 2, 0)
            store_wait(0)
            gather_start(0)

        gather_wait(1)
        store_start(i0 + 1, 1)
        return 0

    jax.lax.fori_loop(0, nch // 2, chunk2, 0)
    gather_wait(0)
    store_wait(1)
    store_start(nch - 1, 0)
    store_wait(0)


def _sc_gather(g, kj_idx):
    mesh = plsc.VectorSubcoreMesh(core_axis_name="c", subcore_axis_name="s")
    return pl.kernel(
        _sc_gather_body,
        out_type=jax.ShapeDtypeStruct((_NA, 128), jnp.float32),
        mesh=mesh,
        compiler_params=pltpu.CompilerParams(use_tc_tiling_on_sc=False),
        scratch_types=[
            pltpu.VMEM((2, 400), jnp.int32),
            pltpu.VMEM((2, 400, 128), jnp.float32),
            pltpu.SemaphoreType.DMA,
            pltpu.SemaphoreType.DMA,
            pltpu.SemaphoreType.DMA,
            pltpu.SemaphoreType.DMA,
            pltpu.SemaphoreType.DMA,
        ],
    )(g, kj_idx)


def _swish(x):
    return x * jax.nn.sigmoid(x)


def _k1_body(m_ref, e_ref, wm_ref, bm_ref, we_ref, g_ref):
    h = jnp.dot(m_ref[...], wm_ref[...], preferred_element_type=jnp.float32)
    h = _swish(h + bm_ref[...])
    ee = jnp.dot(e_ref[...], we_ref[...], preferred_element_type=jnp.float32)
    g_ref[...] = h * ee


def _k1(m_ji, e_rbf8, w_mkj, b_mkj, w_e8, eb):
    n = m_ji.shape[0]
    grid = (n // eb,)
    return pl.pallas_call(
        _k1_body,
        grid=grid,
        in_specs=[
            pl.BlockSpec((eb, 128), lambda i: (i, 0)),
            pl.BlockSpec((eb, 8), lambda i: (i, 0)),
            pl.BlockSpec((128, 128), lambda i: (0, 0)),
            pl.BlockSpec((1, 128), lambda i: (0, 0)),
            pl.BlockSpec((8, 128), lambda i: (0, 0)),
        ],
        out_specs=pl.BlockSpec((eb, 128), lambda i: (i, 0)),
        out_shape=jax.ShapeDtypeStruct((n, 128), jnp.float32),
    )(m_ji, e_rbf8, w_mkj, b_mkj.reshape(1, 128), w_e8)


def _k2_body(a_ref, gm_ref, wa_ref, wb_ref, o_ref):
    a = jnp.dot(a_ref[...], wa_ref[...], preferred_element_type=jnp.float32)
    gm = gm_ref[...]
    acc = jnp.dot(gm * a[:, 0:1], wb_ref[0], preferred_element_type=jnp.float32)
    for j in range(1, 8):
        acc = acc + jnp.dot(gm * a[:, j:j + 1], wb_ref[j],
                            preferred_element_type=jnp.float32)
    o_ref[...] = acc


def _k2(a_sbf, gm, w_a, w_bil2, wb, n):
    grid = (n // wb,)
    gmax = gm.shape[0] // wb - 1
    return pl.pallas_call(
        _k2_body,
        grid=grid,
        in_specs=[
            pl.BlockSpec((wb, 42), lambda i: (jnp.minimum(i, gmax), 0)),
            pl.BlockSpec((wb, 128), lambda i: (jnp.minimum(i, gmax), 0)),
            pl.BlockSpec((42, 8), lambda i: (0, 0)),
            pl.BlockSpec((8, 128, 128), lambda i: (0, 0, 0)),
        ],
        out_specs=pl.BlockSpec((wb, 128), lambda i: (i, 0)),
        out_shape=jax.ShapeDtypeStruct((n, 128), jnp.float32),
    )(a_sbf, gm, w_a, w_bil2)


def _k3_body(m_ref, d_ref, w_ref, b_ref, o_ref):
    m = m_ref[...]
    x = d_ref[...] + _swish(
        jnp.dot(m, w_ref[0], preferred_element_type=jnp.float32) + b_ref[0, 0])
    r = _swish(jnp.dot(x, w_ref[1], preferred_element_type=jnp.float32) + b_ref[0, 1])
    r = _swish(jnp.dot(r, w_ref[2], preferred_element_type=jnp.float32) + b_ref[0, 2])
    x = r + x
    x = _swish(jnp.dot(x, w_ref[3], preferred_element_type=jnp.float32) + b_ref[0, 3]) + m
    r = _swish(jnp.dot(x, w_ref[4], preferred_element_type=jnp.float32) + b_ref[0, 4])
    r = _swish(jnp.dot(r, w_ref[5], preferred_element_type=jnp.float32) + b_ref[0, 5])
    x = r + x
    r = _swish(jnp.dot(x, w_ref[6], preferred_element_type=jnp.float32) + b_ref[0, 6])
    r = _swish(jnp.dot(r, w_ref[7], preferred_element_type=jnp.float32) + b_ref[0, 7])
    o_ref[...] = r + x


def _k3(m_ji, directed, ws, bs, eb):
    n = m_ji.shape[0]
    grid = (n // eb,)
    return pl.pallas_call(
        _k3_body,
        grid=grid,
        in_specs=[
            pl.BlockSpec((eb, 128), lambda i: (i, 0)),
            pl.BlockSpec((eb, 128), lambda i: (i, 0)),
            pl.BlockSpec((8, 128, 128), lambda i: (0, 0, 0)),
            pl.BlockSpec((1, 8, 128), lambda i: (0, 0, 0)),
        ],
        out_specs=pl.BlockSpec((eb, 128), lambda i: (i, 0)),
        out_shape=jax.ShapeDtypeStruct((n, 128), jnp.float32),
    )(m_ji, directed, ws, bs)


def kernel(m_ji, nbr_list, angle_list, e_rbf, a_sbf, kj_idx, ji_idx,
           w_mkj, b_mkj, w_e, w_a, w_bil,
           res0_w0, res0_b0, res0_w1, res0_b1,
           res1_w0, res1_b0, res1_w1, res1_b1,
           res2_w0, res2_b0, res2_w1, res2_b1,
           w_mji, b_mji, w_post, b_post):
    n_edges = m_ji.shape[0]

    e_rbf8 = jnp.pad(e_rbf, ((0, 0), (0, 2)))
    w_e8 = jnp.pad(w_e, ((0, 2), (0, 0)))
    w_bil2 = jnp.transpose(w_bil, (1, 2, 0))

    g = _k1(m_ji, e_rbf8, w_mkj, b_mkj, w_e8, eb=1600)

    npad = _NAP - a_sbf.shape[0]
    ji_p = jnp.pad(ji_idx, (0, npad), constant_values=2**30)
    gm = _sc_gather(g, kj_idx)
    aggr = _k2(a_sbf, gm, w_a, w_bil2, wb=1280, n=_NAP)
    directed = _sc_scatter(aggr, ji_p)

    ws = jnp.stack([w_mji, res0_w0, res0_w1, w_post,
                    res1_w0, res1_w1, res2_w0, res2_w1])
    bs = jnp.stack([b_mji, res0_b0, res0_b1, b_post,
                    res1_b0, res1_b1, res2_b0, res2_b1]).reshape(1, 8, 128)
    return _k3(m_ji, directed, ws, bs, eb=1600)
